# baseline (speedup 1.0000x reference)
import functools

import jax
import jax.numpy as jnp
from jax import lax
from jax.experimental import pallas as pl
from jax.experimental.pallas import tpu as pltpu
from jax.experimental.pallas import tpu_sc as plsc

_V = 1_000_000
_OOV = 1_000
_N = 16384 * 200
_NC, _NS, _L = 2, 16, 16
_NW = _NC * _NS
_PER_TILE = _N // _NW
_OOV_PER_TILE = 640
_EXT = _V + _NS * _OOV_PER_TILE
_BLK = 512
_RCH = 128
_RROWS = _BLK // _RCH
_NBLK = _PER_TILE // _BLK
_ZCH = 2048
_STAGE = 25_000
_NCHUNK = _V // _STAGE


def _lookup_kernel(map_hbm, idx_hbm, out_hbm, table, stage_buf, idx_buf,
                   val_buf, ramp, zeros_buf, oov_buf, sem):
    cid = lax.axis_index("c")
    sid = lax.axis_index("s")
    sid32 = sid.astype(jnp.int32)
    lane = lax.iota(jnp.int32, _L)

    for r in range((_NCHUNK + _NS - 1) // _NS):
        chunk = sid32 + jnp.int32(r * _NS)

        @pl.when(chunk < _NCHUNK)
        def _():
            off = pl.multiple_of(chunk * jnp.int32(_STAGE), 8)
            pltpu.sync_copy(map_hbm.at[pl.ds(off, _STAGE)], stage_buf)
            pltpu.sync_copy(stage_buf, table.at[pl.ds(off, _STAGE)])

    def _oov_step(v, carry):
        o_vec = (sid32 * jnp.int32(_OOV_PER_TILE) + v * jnp.int32(_L) + lane)
        oov_buf[pl.ds(v * jnp.int32(_L), _L)] = jnp.int32(_V) + lax.rem(
            o_vec, jnp.int32(_OOV))
        return carry
    lax.fori_loop(jnp.int32(0), jnp.int32(_OOV_PER_TILE // _L), _oov_step, 0)
    pltpu.sync_copy(oov_buf, table.at[pl.ds(_V + sid32 * jnp.int32(
        _OOV_PER_TILE), _OOV_PER_TILE)])

    for j in range(_RROWS):
        for w in range(_RCH // _L):
            ramp[j, pl.ds(w * _L, _L)] = jnp.int32(
                2 * (_RCH * j + _L * w)) + lane * jnp.int32(2)
    zeros = jnp.zeros((_L,), jnp.int32)

    def _zfill(z, carry):
        zeros_buf[pl.ds(z * jnp.int32(_L), _L)] = zeros
        return carry
    lax.fori_loop(jnp.int32(0), jnp.int32(_ZCH // _L), _zfill, 0)

    base2 = (sid32 * jnp.int32(_NC) + cid.astype(jnp.int32)) * jnp.int32(
        2 * _PER_TILE)

    def _zero_out(z, carry):
        pltpu.sync_copy(zeros_buf,
                        out_hbm.at[pl.ds(base2 + z * jnp.int32(_ZCH), _ZCH)])
        return carry
    lax.fori_loop(jnp.int32(0), jnp.int32(2 * _PER_TILE // _ZCH),
                  _zero_out, 0)
    plsc.subcore_barrier()

    def _blk_step(g, carry):
        off2 = base2 + g * jnp.int32(2 * _BLK)
        win_in = idx_hbm.at[pl.ds(off2, 2 * _BLK)]
        cps = [
            pltpu.async_copy(
                win_in.at[ramp.at[jnp.int32(j)]],
                idx_buf.at[pl.ds(j * _RCH, _RCH)],
                sem,
            )
            for j in range(_RROWS)
        ]
        for c in cps:
            c.wait()
        pltpu.async_copy(table.at[idx_buf], val_buf, sem).wait()
        win_out = out_hbm.at[pl.ds(off2, 2 * _BLK)]
        cps = [
            pltpu.async_copy(
                val_buf.at[pl.ds(j * _RCH, _RCH)],
                win_out.at[ramp.at[jnp.int32(j)]],
                sem,
            )
            for j in range(_RROWS)
        ]
        for c in cps:
            c.wait()
        return carry
    lax.fori_loop(jnp.int32(0), jnp.int32(_NBLK), _blk_step, 0)


@jax.jit
def _lookup(map32, idx_flat):
    mesh = plsc.VectorSubcoreMesh(core_axis_name="c", subcore_axis_name="s")
    return pl.kernel(
        _lookup_kernel,
        out_type=jax.ShapeDtypeStruct((2 * _N,), jnp.int32),
        mesh=mesh,
        scratch_types=[
            pltpu.VMEM_SHARED((_EXT,), jnp.int32),
            pltpu.VMEM((_STAGE,), jnp.int32),
            pltpu.VMEM((_BLK,), jnp.int32),
            pltpu.VMEM((_BLK,), jnp.int32),
            pltpu.VMEM((_RROWS, _RCH), jnp.int32),
            pltpu.VMEM((_ZCH,), jnp.int32),
            pltpu.VMEM((_OOV_PER_TILE,), jnp.int32),
            pltpu.SemaphoreType.DMA,
        ],
        compiler_params=pltpu.CompilerParams(needs_layout_passes=False),
    )(map32, idx_flat)


def kernel(input_text, mapping):
    idx_flat = lax.bitcast_convert_type(input_text, jnp.int32).reshape(2 * _N)
    map32 = mapping.astype(jnp.int32)
    out_flat = _lookup(map32, idx_flat)
    return lax.bitcast_convert_type(
        out_flat.reshape(*input_text.shape, 2), jnp.int64)


# speedup vs baseline: 2.2354x; 2.2354x over previous
import functools

import jax
import jax.numpy as jnp
from jax import lax
from jax.experimental import pallas as pl
from jax.experimental.pallas import tpu as pltpu
from jax.experimental.pallas import tpu_sc as plsc

_V = 1_000_000
_OOV = 1_000
_N = 16384 * 200
_NC, _NS, _L = 2, 16, 16
_NW = _NC * _NS
_PER_TILE = _N // _NW
_OOV_PER_TILE = 640
_EXT = _V + _NS * _OOV_PER_TILE
_BLK = 512
_NBLK = _PER_TILE // _BLK
_STAGE = 25_000
_NCHUNK = _V // _STAGE


def _lookup_kernel(map_hbm, idx_hbm, out_hbm, table, stage_buf, idx_buf,
                   val_buf, oov_buf, sem):
    cid = lax.axis_index("c")
    sid = lax.axis_index("s")
    sid32 = sid.astype(jnp.int32)
    lane = lax.iota(jnp.int32, _L)

    for r in range((_NCHUNK + _NS - 1) // _NS):
        chunk = sid32 + jnp.int32(r * _NS)

        @pl.when(chunk < _NCHUNK)
        def _():
            off = pl.multiple_of(chunk * jnp.int32(_STAGE), 8)
            pltpu.sync_copy(map_hbm.at[pl.ds(off, _STAGE)], stage_buf)
            pltpu.sync_copy(stage_buf, table.at[pl.ds(off, _STAGE)])

    def _oov_step(v, carry):
        o_vec = (sid32 * jnp.int32(_OOV_PER_TILE) + v * jnp.int32(_L) + lane)
        oov_buf[pl.ds(v * jnp.int32(_L), _L)] = jnp.int32(_V) + lax.rem(
            o_vec, jnp.int32(_OOV))
        return carry
    lax.fori_loop(jnp.int32(0), jnp.int32(_OOV_PER_TILE // _L), _oov_step, 0)
    pltpu.sync_copy(oov_buf, table.at[pl.ds(_V + sid32 * jnp.int32(
        _OOV_PER_TILE), _OOV_PER_TILE)])

    plsc.subcore_barrier()

    base = (sid32 * jnp.int32(_NC) + cid.astype(jnp.int32)) * jnp.int32(
        _PER_TILE)

    def _blk_step(g, carry):
        off = base + g * jnp.int32(_BLK)
        pltpu.sync_copy(idx_hbm.at[pl.ds(off, _BLK)], idx_buf)
        pltpu.async_copy(table.at[idx_buf], val_buf, sem).wait()
        pltpu.sync_copy(val_buf, out_hbm.at[pl.ds(off, _BLK)])
        return carry
    lax.fori_loop(jnp.int32(0), jnp.int32(_NBLK), _blk_step, 0)


@jax.jit
def _lookup(map32, idx_words):
    mesh = plsc.VectorSubcoreMesh(core_axis_name="c", subcore_axis_name="s")
    return pl.kernel(
        _lookup_kernel,
        out_type=jax.ShapeDtypeStruct((_N,), jnp.int32),
        mesh=mesh,
        scratch_types=[
            pltpu.VMEM_SHARED((_EXT,), jnp.int32),
            pltpu.VMEM((_STAGE,), jnp.int32),
            pltpu.VMEM((_BLK,), jnp.int32),
            pltpu.VMEM((_BLK,), jnp.int32),
            pltpu.VMEM((_OOV_PER_TILE,), jnp.int32),
            pltpu.SemaphoreType.DMA,
        ],
        compiler_params=pltpu.CompilerParams(needs_layout_passes=False),
    )(map32, idx_words)


def kernel(input_text, mapping):
    words_in = lax.bitcast_convert_type(input_text, jnp.int32).reshape(2 * _N)
    idx32 = lax.slice(words_in, (0,), (2 * _N,), (2,))
    map32 = mapping.astype(jnp.int32)
    out32 = _lookup(map32, idx32)
    out_words = jnp.stack([out32, jnp.zeros_like(out32)], axis=-1)
    return lax.bitcast_convert_type(
        out_words.reshape(*input_text.shape, 2), jnp.int64)


# speedup vs baseline: 2.3994x; 1.0733x over previous
import functools

import jax
import jax.numpy as jnp
from jax import lax
from jax.experimental import pallas as pl
from jax.experimental.pallas import tpu as pltpu
from jax.experimental.pallas import tpu_sc as plsc

_V = 1_000_000
_OOV = 1_000
_N = 16384 * 200
_NC, _NS, _L = 2, 16, 16
_NW = _NC * _NS
_PER_TILE = _N // _NW
_OOV_PER_TILE = 640
_EXT = _V + _NS * _OOV_PER_TILE
_BLK = 512
_NBLK = _PER_TILE // _BLK
_STAGE = 25_000
_NCHUNK = _V // _STAGE


def _lookup_kernel(map_hbm, idx_hbm, out_hbm, table, stage_buf, idx_buf,
                   val_buf, oov_buf, sem):
    cid = lax.axis_index("c")
    sid = lax.axis_index("s")
    sid32 = sid.astype(jnp.int32)
    lane = lax.iota(jnp.int32, _L)

    for r in range((_NCHUNK + _NS - 1) // _NS):
        chunk = sid32 + jnp.int32(r * _NS)

        @pl.when(chunk < _NCHUNK)
        def _():
            off = pl.multiple_of(chunk * jnp.int32(_STAGE), 8)
            pltpu.sync_copy(map_hbm.at[pl.ds(off, _STAGE)], stage_buf)
            pltpu.sync_copy(stage_buf, table.at[pl.ds(off, _STAGE)])

    def _oov_step(v, carry):
        o_vec = (sid32 * jnp.int32(_OOV_PER_TILE) + v * jnp.int32(_L) + lane)
        oov_buf[pl.ds(v * jnp.int32(_L), _L)] = jnp.int32(_V) + lax.rem(
            o_vec, jnp.int32(_OOV))
        return carry
    lax.fori_loop(jnp.int32(0), jnp.int32(_OOV_PER_TILE // _L), _oov_step, 0)
    pltpu.sync_copy(oov_buf, table.at[pl.ds(_V + sid32 * jnp.int32(
        _OOV_PER_TILE), _OOV_PER_TILE)])

    plsc.subcore_barrier()

    base = (sid32 * jnp.int32(_NC) + cid.astype(jnp.int32)) * jnp.int32(
        _PER_TILE)

    def _blk_step(g, carry):
        off = base + g * jnp.int32(_BLK)
        pltpu.sync_copy(idx_hbm.at[pl.ds(off, _BLK)], idx_buf)
        pltpu.async_copy(table.at[idx_buf], val_buf, sem).wait()
        pltpu.sync_copy(val_buf, out_hbm.at[pl.ds(off, _BLK)])
        return carry
    lax.fori_loop(jnp.int32(0), jnp.int32(_NBLK), _blk_step, 0)


@jax.jit
def _lookup(map32, idx_words):
    mesh = plsc.VectorSubcoreMesh(core_axis_name="c", subcore_axis_name="s")
    return pl.kernel(
        _lookup_kernel,
        out_type=jax.ShapeDtypeStruct((_N,), jnp.int32),
        mesh=mesh,
        scratch_types=[
            pltpu.VMEM_SHARED((_EXT,), jnp.int32),
            pltpu.VMEM((_STAGE,), jnp.int32),
            pltpu.VMEM((_BLK,), jnp.int32),
            pltpu.VMEM((_BLK,), jnp.int32),
            pltpu.VMEM((_OOV_PER_TILE,), jnp.int32),
            pltpu.SemaphoreType.DMA,
        ],
        compiler_params=pltpu.CompilerParams(needs_layout_passes=False),
    )(map32, idx_words)


def kernel(input_text, mapping):
    words_in = lax.bitcast_convert_type(input_text, jnp.int32).reshape(2 * _N)
    idx32 = lax.slice(words_in, (0,), (2 * _N,), (2,))
    map32 = mapping.astype(jnp.int32)
    out32 = _lookup(map32, idx32)
    return out32.reshape(input_text.shape).astype(jnp.int64)


# speedup vs baseline: 18.9511x; 7.8984x over previous
import functools

import jax
import jax.numpy as jnp
from jax import lax
from jax.experimental import pallas as pl
from jax.experimental.pallas import tpu as pltpu
from jax.experimental.pallas import tpu_sc as plsc

_V = 1_000_000
_OOV = 1_000
_N = 16384 * 200
_NC, _NS, _L = 2, 16, 16
_NW = _NC * _NS
_PER_TILE = _N // _NW
_OOV_PER_TILE = 640
_EXT = _V + _NS * _OOV_PER_TILE
_BLK = 512
_NBLK = _PER_TILE // _BLK
_STAGE = 25_000
_NCHUNK = _V // _STAGE


def _lookup_kernel(map_hbm, idx_hbm, out_hbm, table, stage_buf, idx_buf,
                   val_buf, oov_buf, sem):
    cid = lax.axis_index("c")
    sid = lax.axis_index("s")
    sid32 = sid.astype(jnp.int32)
    lane = lax.iota(jnp.int32, _L)

    for r in range((_NCHUNK + _NS - 1) // _NS):
        chunk = sid32 + jnp.int32(r * _NS)

        @pl.when(chunk < _NCHUNK)
        def _():
            off = pl.multiple_of(chunk * jnp.int32(_STAGE), 8)
            pltpu.sync_copy(map_hbm.at[pl.ds(off, _STAGE)], stage_buf)
            pltpu.sync_copy(stage_buf, table.at[pl.ds(off, _STAGE)])

    def _oov_step(v, carry):
        o_vec = (sid32 * jnp.int32(_OOV_PER_TILE) + v * jnp.int32(_L) + lane)
        oov_buf[pl.ds(v * jnp.int32(_L), _L)] = jnp.int32(_V) + lax.rem(
            o_vec, jnp.int32(_OOV))
        return carry
    lax.fori_loop(jnp.int32(0), jnp.int32(_OOV_PER_TILE // _L), _oov_step, 0)
    pltpu.sync_copy(oov_buf, table.at[pl.ds(_V + sid32 * jnp.int32(
        _OOV_PER_TILE), _OOV_PER_TILE)])

    plsc.subcore_barrier()

    base = (sid32 * jnp.int32(_NC) + cid.astype(jnp.int32)) * jnp.int32(
        _PER_TILE)

    def _blk_step(g, carry):
        off = base + g * jnp.int32(_BLK)
        pltpu.sync_copy(idx_hbm.at[pl.ds(off, _BLK)], idx_buf)
        pltpu.async_copy(table.at[idx_buf], val_buf, sem).wait()
        pltpu.sync_copy(val_buf, out_hbm.at[pl.ds(off, _BLK)])
        return carry
    lax.fori_loop(jnp.int32(0), jnp.int32(_NBLK), _blk_step, 0)


@jax.jit
def _lookup(map32, idx_words):
    mesh = plsc.VectorSubcoreMesh(core_axis_name="c", subcore_axis_name="s")
    return pl.kernel(
        _lookup_kernel,
        out_type=jax.ShapeDtypeStruct((_N,), jnp.int32),
        mesh=mesh,
        scratch_types=[
            pltpu.VMEM_SHARED((_EXT,), jnp.int32),
            pltpu.VMEM((_STAGE,), jnp.int32),
            pltpu.VMEM((_BLK,), jnp.int32),
            pltpu.VMEM((_BLK,), jnp.int32),
            pltpu.VMEM((_OOV_PER_TILE,), jnp.int32),
            pltpu.SemaphoreType.DMA,
        ],
        compiler_params=pltpu.CompilerParams(needs_layout_passes=False),
    )(map32, idx_words)


def kernel(input_text, mapping):
    idx32 = input_text.astype(jnp.int32).reshape(_N)
    map32 = mapping.astype(jnp.int32)
    out32 = _lookup(map32, idx32)
    return out32.reshape(input_text.shape).astype(jnp.int64)


# speedup vs baseline: 20.8757x; 1.1016x over previous
import functools

import jax
import jax.numpy as jnp
from jax import lax
from jax.experimental import pallas as pl
from jax.experimental.pallas import tpu as pltpu
from jax.experimental.pallas import tpu_sc as plsc

_V = 1_000_000
_OOV = 1_000
_N = 16384 * 200
_NC, _NS, _L = 2, 16, 16
_NW = _NC * _NS
_PER_TILE = _N // _NW
_OOV_PER_TILE = 640
_EXT = _V + _NS * _OOV_PER_TILE
_BLK = 1024
_NBLK = _PER_TILE // _BLK
_STAGE = 25_000
_NCHUNK = _V // _STAGE


def _lookup_kernel(map_hbm, idx_hbm, out_hbm, table, stage_buf, idx_buf,
                   val_buf, oov_buf, sem):
    cid = lax.axis_index("c")
    sid = lax.axis_index("s")
    sid32 = sid.astype(jnp.int32)
    lane = lax.iota(jnp.int32, _L)

    for r in range((_NCHUNK + _NS - 1) // _NS):
        chunk = sid32 + jnp.int32(r * _NS)

        @pl.when(chunk < _NCHUNK)
        def _():
            off = pl.multiple_of(chunk * jnp.int32(_STAGE), 8)
            pltpu.sync_copy(map_hbm.at[pl.ds(off, _STAGE)], stage_buf)
            pltpu.sync_copy(stage_buf, table.at[pl.ds(off, _STAGE)])

    def _oov_step(v, carry):
        o_vec = (sid32 * jnp.int32(_OOV_PER_TILE) + v * jnp.int32(_L) + lane)
        oov_buf[pl.ds(v * jnp.int32(_L), _L)] = jnp.int32(_V) + lax.rem(
            o_vec, jnp.int32(_OOV))
        return carry
    lax.fori_loop(jnp.int32(0), jnp.int32(_OOV_PER_TILE // _L), _oov_step, 0)
    pltpu.sync_copy(oov_buf, table.at[pl.ds(_V + sid32 * jnp.int32(
        _OOV_PER_TILE), _OOV_PER_TILE)])

    plsc.subcore_barrier()

    base = (sid32 * jnp.int32(_NC) + cid.astype(jnp.int32)) * jnp.int32(
        _PER_TILE)

    def _blk_step(g, carry):
        off = base + g * jnp.int32(_BLK)
        pltpu.sync_copy(idx_hbm.at[pl.ds(off, _BLK)], idx_buf)
        pltpu.async_copy(table.at[idx_buf], val_buf, sem).wait()
        pltpu.sync_copy(val_buf, out_hbm.at[pl.ds(off, _BLK)])
        return carry
    lax.fori_loop(jnp.int32(0), jnp.int32(_NBLK), _blk_step, 0)


@jax.jit
def _lookup(map32, idx_words):
    mesh = plsc.VectorSubcoreMesh(core_axis_name="c", subcore_axis_name="s")
    return pl.kernel(
        _lookup_kernel,
        out_type=jax.ShapeDtypeStruct((_N,), jnp.int32),
        mesh=mesh,
        scratch_types=[
            pltpu.VMEM_SHARED((_EXT,), jnp.int32),
            pltpu.VMEM((_STAGE,), jnp.int32),
            pltpu.VMEM((_BLK,), jnp.int32),
            pltpu.VMEM((_BLK,), jnp.int32),
            pltpu.VMEM((_OOV_PER_TILE,), jnp.int32),
            pltpu.SemaphoreType.DMA,
        ],
        compiler_params=pltpu.CompilerParams(needs_layout_passes=False),
    )(map32, idx_words)


def kernel(input_text, mapping):
    idx32 = input_text.astype(jnp.int32).reshape(_N)
    map32 = mapping.astype(jnp.int32)
    out32 = _lookup(map32, idx32)
    return out32.reshape(input_text.shape).astype(jnp.int64)
